# baseline (speedup 1.0000x reference)
import functools

import jax
import jax.numpy as jnp
from jax import lax
from jax.experimental import pallas as pl
from jax.experimental.pallas import tpu as pltpu
from jax.experimental.pallas import tpu_sc as plsc

NUM_EMB_ROWS = 1_000_000
NUM_IDX = 819_200
NC = 2
NS = 16
NW = NC * NS
LANES = 16

NEP = 1 << 20
IDX_COLS = 128
IDX_ROWS_TOTAL = NUM_IDX // IDX_COLS
ROWS_PER_T = IDX_ROWS_TOTAL // NW
IDXB = 8
NIDXB = ROWS_PER_T // IDXB
SP_SLICE = NEP // NS
ZB = 8192

TROWS = 31_232
RCHUNK = 2_048
NFULL = TROWS // RCHUNK
RTAIL = TROWS - NFULL * RCHUNK
REX1_RO = NW * TROWS
REX1_NR = 512
LAST64_RO = REX1_RO + REX1_NR
LAST64 = NUM_EMB_ROWS - LAST64_RO


def _make_mesh():
    return plsc.VectorSubcoreMesh(core_axis_name="c", subcore_axis_name="s")


def _hist_call(ebi2d):
    @functools.partial(
        pl.kernel,
        out_type=(jax.ShapeDtypeStruct((NEP,), jnp.float32),
                  jax.ShapeDtypeStruct((NEP,), jnp.float32)),
        mesh=_make_mesh(),
        scratch_types=[
            pltpu.VMEM_SHARED((NEP,), jnp.float32),
            [pltpu.VMEM((IDXB, IDX_COLS), jnp.int32) for _ in range(2)],
            pltpu.VMEM((IDX_COLS,), jnp.float32),
            pltpu.VMEM((ZB,), jnp.float32),
            pltpu.SemaphoreType.DMA,
        ],
    )
    def hist_kernel(ebi_hbm, hist0_hbm, hist1_hbm, hist_sp, idxb, ones_v,
                    zbuf, sem):
        c = lax.axis_index("c")
        s = lax.axis_index("s")
        wid = s * NC + c

        def fill_z(i, _):
            zbuf[pl.ds(i * LANES, LANES)] = jnp.zeros((LANES,), jnp.float32)
            return 0

        lax.fori_loop(0, ZB // LANES, fill_z, 0)

        def fill_o(i, _):
            ones_v[pl.ds(i * LANES, LANES)] = jnp.ones((LANES,), jnp.float32)
            return 0

        lax.fori_loop(0, IDX_COLS // LANES, fill_o, 0)

        base_sp = s * SP_SLICE

        def zero_sp(i, _):
            pltpu.sync_copy(zbuf, hist_sp.at[pl.ds(base_sp + i * ZB, ZB)])
            return 0

        lax.fori_loop(0, SP_SLICE // ZB, zero_sp, 0)
        plsc.subcore_barrier()

        row0 = wid * ROWS_PER_T
        chunk_descs = [None] * NIDXB
        for k in range(NIDXB):
            b = k % 2
            if k >= 2:
                for d in chunk_descs[k - 2]:
                    d.wait()
            pltpu.sync_copy(ebi_hbm.at[pl.ds(row0 + k * IDXB, IDXB)], idxb[b])
            chunk_descs[k] = [
                pltpu.async_copy(ones_v, hist_sp.at[idxb[b].at[j]],
                                 sem, add=True)
                for j in range(IDXB)]
        for k in (NIDXB - 2, NIDXB - 1):
            for d in chunk_descs[k]:
                d.wait()
        plsc.subcore_barrier()

        @pl.when(c == 0)
        def _():
            pltpu.sync_copy(hist_sp.at[pl.ds(base_sp, SP_SLICE)],
                            hist0_hbm.at[pl.ds(base_sp, SP_SLICE)])

        @pl.when(c == 1)
        def _():
            pltpu.sync_copy(hist_sp.at[pl.ds(base_sp, SP_SLICE)],
                            hist1_hbm.at[pl.ds(base_sp, SP_SLICE)])

    return hist_kernel(ebi2d)


def _wsum_call(hist0, hist1, wpack, lpack):
    @functools.partial(
        pl.kernel,
        out_type=jax.ShapeDtypeStruct((NW * 6 * LANES,), jnp.float32),
        mesh=_make_mesh(),
        scratch_types=[
            [[pltpu.VMEM((16, RCHUNK), jnp.float32),
              pltpu.VMEM((RCHUNK,), jnp.float32),
              pltpu.VMEM((RCHUNK,), jnp.float32)] for _ in range(2)],
            pltpu.VMEM((6 * LANES,), jnp.float32),
            [pltpu.SemaphoreType.DMA for _ in range(2)],
        ],
        compiler_params=pltpu.CompilerParams(needs_layout_passes=False),
    )
    def wsum_kernel(h0_hbm, h1_hbm, wp_hbm, lp_hbm, out_hbm, bufs, ob, sems):
        c = lax.axis_index("c")
        s = lax.axis_index("s")
        wid = s * NC + c
        rbase = wid * TROWS

        gate = jnp.where(wid == NW - 1, 1.0, 0.0).astype(jnp.float32)
        stages = [(rbase + k * RCHUNK, RCHUNK, None, False) for k in range(NFULL)]
        stages.append((rbase + NFULL * RCHUNK, RTAIL, None, False))
        stages.append((REX1_RO, REX1_NR, gate, False))
        stages.append((LAST64_RO, LAST64, gate, True))

        def fire(sidx, ro, nr, l64):
            wb, hb0, hb1 = bufs[sidx]
            if l64:
                src = lp_hbm.at[pl.ds(0, 16), pl.ds(0, IDX_COLS)]
                dstw = wb.at[pl.ds(0, 16), pl.ds(0, IDX_COLS)]
            else:
                src = wp_hbm.at[pl.ds(0, 16), pl.ds(ro, nr)]
                dstw = wb.at[pl.ds(0, 16), pl.ds(0, nr)]
            return [
                pltpu.async_copy(src, dstw, sems[sidx]),
                pltpu.async_copy(h0_hbm.at[pl.ds(ro, nr)],
                                 hb0.at[pl.ds(0, nr)], sems[sidx]),
                pltpu.async_copy(h1_hbm.at[pl.ds(ro, nr)],
                                 hb1.at[pl.ds(0, nr)], sems[sidx]),
            ]

        def rowgroup_body(sidx, gv):
            wb, hb0, hb1 = bufs[sidx]

            def body(t, accs):
                a = list(accs)
                sl = pl.ds(t * LANES, LANES)
                h = hb0[sl] + hb1[sl]
                if gv is not None:
                    h = h * gv
                for cc in range(3):
                    a[cc] = a[cc] + h * (wb[0 + cc, sl] + wb[6 + cc, sl])
                    a[3 + cc] = a[3 + cc] + h * (wb[3 + cc, sl] + wb[9 + cc, sl])
                return tuple(a)
            return body

        accs = (jnp.zeros((LANES,), jnp.float32),) * 6
        descs = fire(0, *stages[0][:2], stages[0][3])
        for i, (ro, nr, g, l64) in enumerate(stages):
            sidx = i % 2
            nxt = None
            if i + 1 < len(stages):
                nro, nnr, _, nl64 = stages[i + 1]
                nxt = fire(1 - sidx, nro, nnr, nl64)
            for d in descs:
                d.wait()
            accs = lax.fori_loop(0, nr // LANES, rowgroup_body(sidx, g), accs)
            descs = nxt

        for i in range(6):
            ob[pl.ds(i * LANES, LANES)] = accs[i]
        pltpu.sync_copy(ob, out_hbm.at[pl.ds(wid * 6 * LANES, 6 * LANES)])

    return wsum_kernel(hist0, hist1, wpack, lpack)


def kernel(eb_input, eb_offset, W0, W1, W2, W3):
    del eb_offset
    tables = (W0, W1, W2, W3)
    ebi2d = eb_input.reshape(IDX_ROWS_TOTAL, IDX_COLS)
    hist0, hist1 = _hist_call(ebi2d)
    wpack = jnp.concatenate(
        [jnp.swapaxes(W, 0, 1) for W in tables]
        + [jnp.zeros((16 - 3 * len(tables), NUM_EMB_ROWS), jnp.float32)],
        axis=0)
    lpack = jnp.zeros((16, IDX_COLS), jnp.float32).at[:12, :LAST64].set(
        jnp.stack([W[LAST64_RO:, cc] for W in tables for cc in range(3)]))
    partials = _wsum_call(hist0, hist1, wpack, lpack)
    return jnp.sum(partials.reshape(NW, 6, LANES), axis=(0, 2))


# speedup vs baseline: 1.9417x; 1.9417x over previous
import functools

import numpy as np
import jax
import jax.numpy as jnp
from jax import lax
from jax.experimental import pallas as pl
from jax.experimental.pallas import tpu as pltpu
from jax.experimental.pallas import tpu_sc as plsc

NUM_EMB_ROWS = 1_000_000
NUM_IDX = 819_200
NC = 2
NS = 16
NW = NC * NS
LANES = 16

NEP = 1 << 20
IDX_COLS = 128
IDX_ROWS_TOTAL = NUM_IDX // IDX_COLS
ROWS_PER_W = IDX_ROWS_TOTAL // NW
SCAT_AHEAD = 16
SP_SLICE = NEP // NS
ZB = 8192

ROWS_W = 31_248
RCHUNK = 2_608
NRCH = 11
RTAIL = ROWS_W - NRCH * RCHUNK
REXTRA = NUM_EMB_ROWS - NW * ROWS_W

def _make_mesh():
    return plsc.VectorSubcoreMesh(core_axis_name="c", subcore_axis_name="s")


def _hist_call(ebi2d):
    @functools.partial(
        pl.kernel,
        out_type=(jax.ShapeDtypeStruct((NEP,), jnp.float32),
                  jax.ShapeDtypeStruct((NEP,), jnp.float32)),
        mesh=_make_mesh(),
        scratch_types=[
            pltpu.VMEM_SHARED((NEP,), jnp.float32),
            pltpu.VMEM((ROWS_PER_W, IDX_COLS), jnp.int32),
            pltpu.VMEM((IDX_COLS,), jnp.float32),
            pltpu.VMEM((ZB,), jnp.float32),
            pltpu.SemaphoreType.DMA,
        ],
    )
    def hist_kernel(ebi_hbm, hist0_hbm, hist1_hbm, hist_sp, idx_v, ones_v, zbuf, sem):
        c = lax.axis_index("c")
        s = lax.axis_index("s")
        wid = s * NC + c

        def fill_z(i, _):
            zbuf[pl.ds(i * LANES, LANES)] = jnp.zeros((LANES,), jnp.float32)
            return 0

        lax.fori_loop(0, ZB // LANES, fill_z, 0)

        def fill_o(i, _):
            ones_v[pl.ds(i * LANES, LANES)] = jnp.ones((LANES,), jnp.float32)
            return 0

        lax.fori_loop(0, IDX_COLS // LANES, fill_o, 0)

        base_sp = s * SP_SLICE

        def zero_sp(i, _):
            pltpu.sync_copy(zbuf, hist_sp.at[pl.ds(base_sp + i * ZB, ZB)])
            return 0

        lax.fori_loop(0, SP_SLICE // ZB, zero_sp, 0)
        plsc.subcore_barrier()

        row0 = wid * ROWS_PER_W
        pltpu.sync_copy(ebi_hbm.at[pl.ds(row0, ROWS_PER_W)], idx_v)
        scat = []
        for g in range(ROWS_PER_W):
            scat.append(pltpu.async_copy(
                ones_v, hist_sp.at[idx_v.at[g]], sem, add=True))
            if g >= SCAT_AHEAD:
                scat[g - SCAT_AHEAD].wait()
        for g in range(ROWS_PER_W - SCAT_AHEAD, ROWS_PER_W):
            scat[g].wait()
        plsc.subcore_barrier()

        @pl.when(c == 0)
        def _():
            pltpu.sync_copy(hist_sp.at[pl.ds(base_sp, SP_SLICE)],
                            hist0_hbm.at[pl.ds(base_sp, SP_SLICE)])

        @pl.when(c == 1)
        def _():
            pltpu.sync_copy(hist_sp.at[pl.ds(base_sp, SP_SLICE)],
                            hist1_hbm.at[pl.ds(base_sp, SP_SLICE)])

    return hist_kernel(ebi2d)


def _wsum_call(hist0, hist1, wcols):
    @functools.partial(
        pl.kernel,
        out_type=jax.ShapeDtypeStruct((NW * 6 * LANES,), jnp.float32),
        mesh=_make_mesh(),
        scratch_types=[
            [[pltpu.VMEM((RCHUNK,), jnp.float32) for _ in range(14)]
             for _ in range(2)],
            pltpu.VMEM((6 * LANES,), jnp.float32),
            [pltpu.SemaphoreType.DMA for _ in range(2)],
        ],
        compiler_params=pltpu.CompilerParams(needs_layout_passes=False),
    )
    def wsum_kernel(h0_hbm, h1_hbm, *rest):
        wc_hbm = rest[:12]
        out_hbm = rest[12]
        bufs = rest[13]
        ob = rest[14]
        sems = rest[15]
        c = lax.axis_index("c")
        s = lax.axis_index("s")
        wid = s * NC + c
        rbase = wid * ROWS_W

        def fire(sidx, ro, nr):
            ds = []
            for i in range(12):
                ds.append(pltpu.async_copy(
                    wc_hbm[i].at[pl.ds(ro, nr)],
                    bufs[sidx][i].at[pl.ds(0, nr)], sems[sidx]))
            ds.append(pltpu.async_copy(
                h0_hbm.at[pl.ds(ro, nr)], bufs[sidx][12].at[pl.ds(0, nr)],
                sems[sidx]))
            ds.append(pltpu.async_copy(
                h1_hbm.at[pl.ds(ro, nr)], bufs[sidx][13].at[pl.ds(0, nr)],
                sems[sidx]))
            return ds

        def rowgroup_body(sidx, gate):
            wcb = bufs[sidx]

            def body(t, accs):
                a = list(accs)
                sl = pl.ds(t * LANES, LANES)
                h = wcb[12][sl] + wcb[13][sl]
                if gate is not None:
                    h = h * gate
                for cc in range(3):
                    a[cc] = a[cc] + h * (wcb[0 + cc][sl] + wcb[6 + cc][sl])
                    a[3 + cc] = a[3 + cc] + h * (wcb[3 + cc][sl] + wcb[9 + cc][sl])
                return tuple(a)
            return body

        zero16 = jnp.zeros((LANES,), jnp.float32)
        accs = (zero16,) * 6

        gate = jnp.where(wid == NW - 1, 1.0, 0.0).astype(jnp.float32)
        stages = [(rbase + k * RCHUNK, RCHUNK, None) for k in range(NRCH)]
        stages.append((rbase + NRCH * RCHUNK, RTAIL, None))
        stages.append((NUM_EMB_ROWS - REXTRA, REXTRA, gate))

        descs = fire(0, stages[0][0], stages[0][1])
        for i, (ro, nr, g) in enumerate(stages):
            sidx = i % 2
            nxt = None
            if i + 1 < len(stages):
                nxt = fire(1 - sidx, stages[i + 1][0], stages[i + 1][1])
            for d in descs:
                d.wait()
            accs = lax.fori_loop(0, nr // LANES, rowgroup_body(sidx, g), accs)
            descs = nxt

        for i in range(6):
            ob[pl.ds(i * LANES, LANES)] = accs[i]
        pltpu.sync_copy(ob, out_hbm.at[pl.ds(wid * 6 * LANES, 6 * LANES)])

    return wsum_kernel(hist0, hist1, *wcols)


def kernel(eb_input, eb_offset, W0, W1, W2, W3):
    del eb_offset
    ebi2d = eb_input.reshape(IDX_ROWS_TOTAL, IDX_COLS)
    hist0, hist1 = _hist_call(ebi2d)
    wcols = [W[:, cc] for W in (W0, W1, W2, W3) for cc in range(3)]
    partials = _wsum_call(hist0, hist1, wcols)
    return jnp.sum(partials.reshape(NW, 6, LANES), axis=(0, 2))
